# baseline (speedup 1.0000x reference)
import jax
import jax.numpy as jnp
import numpy as np
from jax import lax
from jax.experimental import pallas as pl
from jax.experimental.pallas import tpu as pltpu
from jax.experimental.pallas import tpu_sc as plsc

NOTES_POOL_SIZE = 1000000
EMBED_DIM = 64
BATCH = 4096
SEQ_LEN = 200

NC = 2
NS = 16
NW = NC * NS
ROWS_PER_W = BATCH // NW
NCHUNK = 5
CHUNK = SEQ_LEN // NCHUNK


def _positional_encoding(max_pos, embed_dim):
    pos = np.arange(max_pos)[:, np.newaxis]
    i = np.arange(embed_dim)[np.newaxis, :]
    angle_rates = 1.0 / np.power(10000, 2 * (i // 2) / np.float32(embed_dim))
    angle_rads = pos * angle_rates
    angle_rads[:, 0::2] = np.sin(angle_rads[:, 0::2])
    angle_rads[:, 1::2] = np.cos(angle_rads[:, 1::2])
    return angle_rads.astype(np.float32)


def _body(x_hbm, table_hbm, pos_hbm, out_hbm, idx_v, rows_v, pos_v, gsem):
    wid = lax.axis_index("s") * NC + lax.axis_index("c")
    pltpu.sync_copy(pos_hbm, pos_v)

    def row(b, carry):
        bb = wid * ROWS_PER_W + b
        pltpu.sync_copy(x_hbm.at[bb], idx_v)
        copies = [
            pltpu.async_copy(table_hbm.at[idx_v.at[h]], rows_v.at[h], gsem)
            for h in range(NCHUNK)
        ]
        for c in copies:
            c.wait()

        for h in range(NCHUNK):
            def inner(i, _, h=h):
                for j in range(EMBED_DIM // 16):
                    s = pl.ds(16 * j, 16)
                    rows_v[h, i, s] = rows_v[h, i, s] + pos_v[h, i, s]
                return 0
            lax.fori_loop(0, CHUNK, inner, 0)

        pltpu.sync_copy(rows_v, out_hbm.at[bb])
        return carry

    lax.fori_loop(0, ROWS_PER_W, row, 0)


@jax.jit
def _run(x3, table, pos):
    mesh = plsc.VectorSubcoreMesh(core_axis_name="c", subcore_axis_name="s")
    k = pl.kernel(
        _body,
        out_type=jax.ShapeDtypeStruct((BATCH, NCHUNK, CHUNK, EMBED_DIM),
                                      jnp.float32),
        mesh=mesh,
        scratch_types=[
            pltpu.VMEM((NCHUNK, CHUNK), jnp.int32),
            pltpu.VMEM((NCHUNK, CHUNK, EMBED_DIM), jnp.float32),
            pltpu.VMEM((NCHUNK, CHUNK, EMBED_DIM), jnp.float32),
            pltpu.SemaphoreType.DMA,
        ],
        compiler_params=pltpu.CompilerParams(use_tc_tiling_on_sc=False),
    )
    return k(x3, table, pos)


def kernel(x_in, table):
    x3 = x_in.astype(jnp.int32).reshape(BATCH, NCHUNK, CHUNK)
    pos = jnp.asarray(
        _positional_encoding(SEQ_LEN, EMBED_DIM).reshape(
            NCHUNK, CHUNK, EMBED_DIM))
    out = _run(x3, table, pos)
    return out.reshape(BATCH, SEQ_LEN, EMBED_DIM)


# speedup vs baseline: 1.1741x; 1.1741x over previous
import jax
import jax.numpy as jnp
import numpy as np
from jax import lax
from jax.experimental import pallas as pl
from jax.experimental.pallas import tpu as pltpu
from jax.experimental.pallas import tpu_sc as plsc

NOTES_POOL_SIZE = 1000000
EMBED_DIM = 64
BATCH = 4096
SEQ_LEN = 200

NC = 2
NS = 16
NW = NC * NS
ROWS_PER_W = BATCH // NW
NCHUNK = 5
CHUNK = SEQ_LEN // NCHUNK
NBUF = 3


def _positional_encoding(max_pos, embed_dim):
    pos = np.arange(max_pos)[:, np.newaxis]
    i = np.arange(embed_dim)[np.newaxis, :]
    angle_rates = 1.0 / np.power(10000, 2 * (i // 2) / np.float32(embed_dim))
    angle_rads = pos * angle_rates
    angle_rads[:, 0::2] = np.sin(angle_rads[:, 0::2])
    angle_rads[:, 1::2] = np.cos(angle_rads[:, 1::2])
    return angle_rads.astype(np.float32)


def _body(x_hbm, table_hbm, pos_hbm, out_hbm,
          pos_v, idx_v, rows0, rows1, rows2,
          psem, xsem, gsem0, gsem1, gsem2, ssem0, ssem1, ssem2):
    rows = [rows0, rows1, rows2]
    gsem = [gsem0, gsem1, gsem2]
    ssem = [ssem0, ssem1, ssem2]

    wid = lax.axis_index("s") * NC + lax.axis_index("c")
    base = wid * ROWS_PER_W

    pltpu.async_copy(pos_hbm, pos_v, psem)
    pltpu.async_copy(x_hbm.at[pl.ds(base, ROWS_PER_W)], idx_v, xsem)
    pltpu.make_async_copy(pos_hbm, pos_v, psem).wait()
    pltpu.make_async_copy(x_hbm.at[pl.ds(base, ROWS_PER_W)], idx_v,
                          xsem).wait()

    def fire_gathers(b, q):
        for h in range(NCHUNK):
            pltpu.async_copy(table_hbm.at[idx_v.at[b, h]],
                             rows[q].at[pl.ds(h * CHUNK, CHUNK)], gsem[q])

    def wait_gather(b, q):
        for h in range(NCHUNK):
            pltpu.make_async_copy(table_hbm.at[idx_v.at[b, h]],
                                  rows[q].at[pl.ds(h * CHUNK, CHUNK)],
                                  gsem[q]).wait()

    def wait_store(q):
        pltpu.make_async_copy(rows[q], out_hbm.at[base], ssem[q]).wait()

    def add_pos(q):
        def inner(i, carry):
            for ii in range(8):
                r = 8 * i + ii
                for j in range(EMBED_DIM // 16):
                    s = pl.ds(16 * j, 16)
                    plsc.addupdate(rows[q].at[r, s], pos_v[r, s])
            return carry
        lax.fori_loop(0, SEQ_LEN // 8, inner, 0)

    fire_gathers(0, 0)

    def step(b, carry):
        q0 = lax.rem(b, NBUF)
        for q in range(NBUF):
            @pl.when(q0 == q)
            def _(q=q):
                wait_gather(b, q)

                @pl.when(b >= 2)
                def _():
                    wait_store((q + 1) % NBUF)

                @pl.when(b < ROWS_PER_W - 1)
                def _():
                    fire_gathers(b + 1, (q + 1) % NBUF)

                add_pos(q)
                pltpu.async_copy(rows[q], out_hbm.at[base + b], ssem[q])
        return carry

    lax.fori_loop(0, ROWS_PER_W, step, 0)

    wait_store((ROWS_PER_W - 2) % NBUF)
    wait_store((ROWS_PER_W - 1) % NBUF)


@jax.jit
def _run(x3, table, pos):
    mesh = plsc.VectorSubcoreMesh(core_axis_name="c", subcore_axis_name="s")
    k = pl.kernel(
        _body,
        out_type=jax.ShapeDtypeStruct((BATCH, SEQ_LEN, EMBED_DIM),
                                      jnp.float32),
        mesh=mesh,
        scratch_types=[
            pltpu.VMEM((SEQ_LEN, EMBED_DIM), jnp.float32),
            pltpu.VMEM((ROWS_PER_W, NCHUNK, CHUNK), jnp.int32),
            pltpu.VMEM((SEQ_LEN, EMBED_DIM), jnp.float32),
            pltpu.VMEM((SEQ_LEN, EMBED_DIM), jnp.float32),
            pltpu.VMEM((SEQ_LEN, EMBED_DIM), jnp.float32),
            pltpu.SemaphoreType.DMA,
            pltpu.SemaphoreType.DMA,
            pltpu.SemaphoreType.DMA,
            pltpu.SemaphoreType.DMA,
            pltpu.SemaphoreType.DMA,
            pltpu.SemaphoreType.DMA,
            pltpu.SemaphoreType.DMA,
            pltpu.SemaphoreType.DMA,
        ],
        compiler_params=pltpu.CompilerParams(use_tc_tiling_on_sc=False),
    )
    return k(x3, table, pos)


def kernel(x_in, table):
    x3 = x_in.astype(jnp.int32).reshape(BATCH, NCHUNK, CHUNK)
    pos = jnp.asarray(_positional_encoding(SEQ_LEN, EMBED_DIM))
    return _run(x3, table, pos)
